# baseline (speedup 1.0000x reference)
import jax
import jax.numpy as jnp
from jax import lax
from jax.experimental import pallas as pl
from jax.experimental.pallas import tpu as pltpu
from jax.experimental.pallas import tpu_sc as plsc

N = 10000
E = 320000
D = 128
H = 128
Z = 64
G = 64

NC, NS, L = 2, 16, 16
NTILES = NC * NS
CW = 128
RPT = 80
EP = NTILES * RPT * CW
NP = 10240
NACC = 10112
SEG = NACC // NS
CHI = 16
A0 = 3
A1 = 7
NU = N + 16

_mesh = plsc.VectorSubcoreMesh(
    core_axis_name="c", subcore_axis_name="s", num_cores=NC, num_subcores=NS)


def _deg_body(dst_hbm, out_hbm, dst_v, deg_v):
    c = lax.axis_index("c")
    s = lax.axis_index("s")
    wid = c * NS + s
    pltpu.sync_copy(dst_hbm.at[pl.ds(wid * RPT, RPT)], dst_v)
    zeros16 = jnp.zeros((L,), jnp.float32)

    def zero_body(i, carry):
        deg_v[pl.ds(i * L, L)] = zeros16
        return carry

    lax.fori_loop(0, NP // L, zero_body, 0)
    ones16 = jnp.ones((L,), jnp.float32)

    def edge_body(j, carry):
        for k in range(CW // L):
            idx = dst_v[j, pl.ds(k * L, L)]
            plsc.addupdate_scatter(deg_v, [idx], ones16)
        return carry

    lax.fori_loop(0, RPT, edge_body, 0)
    pltpu.sync_copy(deg_v, out_hbm.at[wid])


_deg_kernel = pl.kernel(
    _deg_body,
    out_type=jax.ShapeDtypeStruct((NTILES, NP), jnp.float32),
    mesh=_mesh,
    scratch_types=[
        pltpu.VMEM((RPT, CW), jnp.int32),
        pltpu.VMEM((NP,), jnp.float32),
    ],
    compiler_params=pltpu.CompilerParams(needs_layout_passes=False),
)


def _agg_body(u_hbm, src_hbm, dst_hbm, out_hbm,
              src_v, dst_v, bufa, bufb, acc_sh, sema, semb):
    c = lax.axis_index("c")
    s = lax.axis_index("s")
    zeros16 = jnp.zeros((L,), jnp.float32)

    def zb(r, carry):
        for k in range(H // L):
            bufa[r, pl.ds(k * L, L)] = zeros16
        return carry

    lax.fori_loop(0, CW, zb, 0)
    for k in range(SEG // CW):
        pltpu.sync_copy(bufa, acc_sh.at[pl.ds(s * SEG + k * CW, CW)])
    rem = SEG - (SEG // CW) * CW
    if rem:
        pltpu.sync_copy(bufa.at[pl.ds(0, rem)],
                        acc_sh.at[pl.ds(s * SEG + (SEG // CW) * CW, rem)])
    plsc.subcore_barrier()

    def step(j, buf, sem):
        pltpu.make_async_copy(u_hbm.at[src_v.at[j]], buf, sem).wait()
        pltpu.sync_copy(buf, acc_sh.at[dst_v.at[j]], add=True)

    nchunks = jnp.where(c == 0, A0, A1)
    base_chunk = jnp.where(c == 0, s * A0, NS * A0 + s * A1)

    def chunk_body(ci, carry):
        row0 = (base_chunk + ci) * CHI
        pltpu.sync_copy(src_hbm.at[pl.ds(row0, CHI)], src_v)
        pltpu.sync_copy(dst_hbm.at[pl.ds(row0, CHI)], dst_v)
        pltpu.async_copy(u_hbm.at[src_v.at[0]], bufa, sema)

        def body(jj, carry2):
            j0 = 2 * jj
            pltpu.async_copy(u_hbm.at[src_v.at[j0 + 1]], bufb, semb)
            step(j0, bufa, sema)
            pltpu.async_copy(u_hbm.at[src_v.at[j0 + 2]], bufa, sema)
            step(j0 + 1, bufb, semb)
            return carry2

        lax.fori_loop(0, CHI // 2 - 1, body, 0)
        pltpu.async_copy(u_hbm.at[src_v.at[CHI - 1]], bufb, semb)
        step(CHI - 2, bufa, sema)
        step(CHI - 1, bufb, semb)
        return carry

    lax.fori_loop(0, nchunks, chunk_body, 0)
    plsc.subcore_barrier()
    pltpu.sync_copy(acc_sh.at[pl.ds(s * SEG, SEG)],
                    out_hbm.at[c, pl.ds(s * SEG, SEG)])


_agg_kernel = pl.kernel(
    _agg_body,
    out_type=jax.ShapeDtypeStruct((NC, NACC, H), jnp.float32),
    mesh=_mesh,
    scratch_types=[
        pltpu.VMEM((CHI, CW), jnp.int32),
        pltpu.VMEM((CHI, CW), jnp.int32),
        pltpu.VMEM((CW, H), jnp.float32),
        pltpu.VMEM((CW, H), jnp.float32),
        pltpu.VMEM_SHARED((NACC, H), jnp.float32),
        pltpu.SemaphoreType.DMA,
        pltpu.SemaphoreType.DMA,
    ],
    compiler_params=pltpu.CompilerParams(needs_layout_passes=False),
)

_PREC = lax.Precision.HIGHEST


def _tc1_body(degT, x, w1, u1_o, dinv_o):
    deg = jnp.sum(degT[...], axis=1, keepdims=True) + 1.0
    dinv = 1.0 / jnp.sqrt(deg)
    dinv_o[...] = dinv
    u1_o[...] = jnp.dot(x[...] * dinv, w1[...],
                        preferred_element_type=jnp.float32, precision=_PREC)


_tc1 = pl.pallas_call(
    _tc1_body,
    out_shape=[jax.ShapeDtypeStruct((N, H), jnp.float32),
               jax.ShapeDtypeStruct((N, 1), jnp.float32)],
)


def _tc2_body(parts, u1, dinv, b1, w2, u2_o):
    agg = parts[0, :N, :] + parts[1, :N, :] + u1[...]
    h1 = jnp.maximum(dinv[...] * agg + b1[...], 0.0)
    u2_o[...] = jnp.dot(h1 * dinv[...], w2[...],
                        preferred_element_type=jnp.float32, precision=_PREC)


_tc2 = pl.pallas_call(
    _tc2_body,
    out_shape=jax.ShapeDtypeStruct((N, H), jnp.float32),
)


def _tc3_body(parts, u2, dinv, b2, batch_row, eps,
              wmu, bmu, wlv, blv, wd1, bd1, wd2, bd2,
              recon_o, mu_o, lv_o):
    agg = parts[0, :N, :] + parts[1, :N, :] + u2[...]
    h2 = jnp.maximum(dinv[...] * agg + b2[...], 0.0)
    oh = (batch_row[...] == lax.broadcasted_iota(jnp.int32, (G, N), 0)
          ).astype(jnp.float32)
    S = lax.dot_general(oh, h2, (((1,), (0,)), ((), ())),
                        preferred_element_type=jnp.float32, precision=_PREC)
    cnt = jnp.sum(oh, axis=1, keepdims=True)
    g = S / jnp.maximum(cnt, 1.0)
    mu = jnp.dot(g, wmu[...], preferred_element_type=jnp.float32,
                 precision=_PREC) + bmu[...]
    lv = jnp.dot(g, wlv[...], preferred_element_type=jnp.float32,
                 precision=_PREC) + blv[...]
    std = jnp.exp(0.5 * lv)
    z = mu + eps[...] * std
    hr = jnp.maximum(jnp.dot(z, wd1[...], preferred_element_type=jnp.float32,
                             precision=_PREC) + bd1[...], 0.0)
    recon_o[...] = jnp.dot(hr, wd2[...], preferred_element_type=jnp.float32,
                           precision=_PREC) + bd2[...]
    mu_o[...] = mu
    lv_o[...] = lv


_tc3 = pl.pallas_call(
    _tc3_body,
    out_shape=[jax.ShapeDtypeStruct((G, D), jnp.float32),
               jax.ShapeDtypeStruct((G, Z), jnp.float32),
               jax.ShapeDtypeStruct((G, Z), jnp.float32)],
)


def kernel(x, edge_index, batch, eps, W1, b1, W2, b2,
           Wmu, bmu, Wlv, blv, Wd1, bd1, Wd2, bd2):
    src = edge_index[0].astype(jnp.int32)
    dst = edge_index[1].astype(jnp.int32)
    padn = jnp.full((EP - E,), N, jnp.int32)
    srcp = jnp.concatenate([src, padn]).reshape(EP // CW, CW)
    dstp = jnp.concatenate([dst, padn]).reshape(EP // CW, CW)

    deg_parts = _deg_kernel(dstp)
    degT = deg_parts.T[:N]
    u1, dinv = _tc1(degT, x, W1)

    zpad = jnp.zeros((NU - N, H), jnp.float32)
    agg1 = _agg_kernel(jnp.concatenate([u1, zpad]), srcp, dstp)
    u2 = _tc2(agg1, u1, dinv, b1.reshape(1, H), W2)
    agg2 = _agg_kernel(jnp.concatenate([u2, zpad]), srcp, dstp)

    recon, mu, lv = _tc3(
        agg2, u2, dinv, b2.reshape(1, H),
        batch.astype(jnp.int32).reshape(1, N), eps,
        Wmu, bmu.reshape(1, Z), Wlv, blv.reshape(1, Z),
        Wd1, bd1.reshape(1, H), Wd2, bd2.reshape(1, D))
    return (recon, mu, lv)


# speedup vs baseline: 1.0766x; 1.0766x over previous
import jax
import jax.numpy as jnp
from jax import lax
from jax.experimental import pallas as pl
from jax.experimental.pallas import tpu as pltpu
from jax.experimental.pallas import tpu_sc as plsc

N = 10000
E = 320000
D = 128
H = 128
Z = 64
G = 64

NC, NS, L = 2, 16, 16
NTILES = NC * NS
CW = 128
RPT = 80
EP = NTILES * RPT * CW
NP = 10240
NACC = 10112
SEG = NACC // NS
CHI = 16
A0 = 7
A1 = 3
NU = N + 16

_mesh = plsc.VectorSubcoreMesh(
    core_axis_name="c", subcore_axis_name="s", num_cores=NC, num_subcores=NS)


def _deg_body(dst_hbm, out_hbm, dst_v, deg_v):
    c = lax.axis_index("c")
    s = lax.axis_index("s")
    wid = c * NS + s
    pltpu.sync_copy(dst_hbm.at[pl.ds(wid * RPT, RPT)], dst_v)
    zeros16 = jnp.zeros((L,), jnp.float32)

    def zero_body(i, carry):
        deg_v[pl.ds(i * L, L)] = zeros16
        return carry

    lax.fori_loop(0, NP // L, zero_body, 0)
    ones16 = jnp.ones((L,), jnp.float32)

    def edge_body(j, carry):
        for k in range(CW // L):
            idx = dst_v[j, pl.ds(k * L, L)]
            plsc.addupdate_scatter(deg_v, [idx], ones16)
        return carry

    lax.fori_loop(0, RPT, edge_body, 0)
    pltpu.sync_copy(deg_v, out_hbm.at[wid])


_deg_kernel = pl.kernel(
    _deg_body,
    out_type=jax.ShapeDtypeStruct((NTILES, NP), jnp.float32),
    mesh=_mesh,
    scratch_types=[
        pltpu.VMEM((RPT, CW), jnp.int32),
        pltpu.VMEM((NP,), jnp.float32),
    ],
    compiler_params=pltpu.CompilerParams(needs_layout_passes=False),
)


def _agg_body(u_hbm, src_hbm, dst_hbm, out_hbm,
              src_v, dst_v, bufa, bufb, acc_sh, sema, semb):
    c = lax.axis_index("c")
    s = lax.axis_index("s")
    zeros16 = jnp.zeros((L,), jnp.float32)

    def zb(r, carry):
        for k in range(H // L):
            bufa[r, pl.ds(k * L, L)] = zeros16
        return carry

    lax.fori_loop(0, CW, zb, 0)
    for k in range(SEG // CW):
        pltpu.sync_copy(bufa, acc_sh.at[pl.ds(s * SEG + k * CW, CW)])
    rem = SEG - (SEG // CW) * CW
    if rem:
        pltpu.sync_copy(bufa.at[pl.ds(0, rem)],
                        acc_sh.at[pl.ds(s * SEG + (SEG // CW) * CW, rem)])
    plsc.subcore_barrier()

    def step(j, buf, sem):
        pltpu.make_async_copy(u_hbm.at[src_v.at[j]], buf, sem).wait()
        pltpu.sync_copy(buf, acc_sh.at[dst_v.at[j]], add=True)

    nchunks = jnp.where(c == 0, A0, A1)
    base_chunk = jnp.where(c == 0, s * A0, NS * A0 + s * A1)

    def chunk_body(ci, carry):
        row0 = (base_chunk + ci) * CHI
        pltpu.sync_copy(src_hbm.at[pl.ds(row0, CHI)], src_v)
        pltpu.sync_copy(dst_hbm.at[pl.ds(row0, CHI)], dst_v)
        pltpu.async_copy(u_hbm.at[src_v.at[0]], bufa, sema)

        def body(jj, carry2):
            j0 = 2 * jj
            pltpu.async_copy(u_hbm.at[src_v.at[j0 + 1]], bufb, semb)
            step(j0, bufa, sema)
            pltpu.async_copy(u_hbm.at[src_v.at[j0 + 2]], bufa, sema)
            step(j0 + 1, bufb, semb)
            return carry2

        lax.fori_loop(0, CHI // 2 - 1, body, 0)
        pltpu.async_copy(u_hbm.at[src_v.at[CHI - 1]], bufb, semb)
        step(CHI - 2, bufa, sema)
        step(CHI - 1, bufb, semb)
        return carry

    lax.fori_loop(0, nchunks, chunk_body, 0)
    plsc.subcore_barrier()
    pltpu.sync_copy(acc_sh.at[pl.ds(s * SEG, SEG)],
                    out_hbm.at[c, pl.ds(s * SEG, SEG)])


_agg_kernel = pl.kernel(
    _agg_body,
    out_type=jax.ShapeDtypeStruct((NC, NACC, H), jnp.float32),
    mesh=_mesh,
    scratch_types=[
        pltpu.VMEM((CHI, CW), jnp.int32),
        pltpu.VMEM((CHI, CW), jnp.int32),
        pltpu.VMEM((CW, H), jnp.float32),
        pltpu.VMEM((CW, H), jnp.float32),
        pltpu.VMEM_SHARED((NACC, H), jnp.float32),
        pltpu.SemaphoreType.DMA,
        pltpu.SemaphoreType.DMA,
    ],
    compiler_params=pltpu.CompilerParams(needs_layout_passes=False),
)

_PREC = lax.Precision.HIGHEST


def _tc1_body(degT, x, w1, u1_o, dinv_o):
    deg = jnp.sum(degT[...], axis=1, keepdims=True) + 1.0
    dinv = 1.0 / jnp.sqrt(deg)
    dinv_o[...] = dinv
    u1_o[...] = jnp.dot(x[...] * dinv, w1[...],
                        preferred_element_type=jnp.float32, precision=_PREC)


_tc1 = pl.pallas_call(
    _tc1_body,
    out_shape=[jax.ShapeDtypeStruct((N, H), jnp.float32),
               jax.ShapeDtypeStruct((N, 1), jnp.float32)],
)


def _tc2_body(parts, u1, dinv, b1, w2, u2_o):
    agg = parts[0, :N, :] + parts[1, :N, :] + u1[...]
    h1 = jnp.maximum(dinv[...] * agg + b1[...], 0.0)
    u2_o[...] = jnp.dot(h1 * dinv[...], w2[...],
                        preferred_element_type=jnp.float32, precision=_PREC)


_tc2 = pl.pallas_call(
    _tc2_body,
    out_shape=jax.ShapeDtypeStruct((N, H), jnp.float32),
)


def _tc3_body(parts, u2, dinv, b2, batch_row, eps,
              wmu, bmu, wlv, blv, wd1, bd1, wd2, bd2,
              recon_o, mu_o, lv_o):
    agg = parts[0, :N, :] + parts[1, :N, :] + u2[...]
    h2 = jnp.maximum(dinv[...] * agg + b2[...], 0.0)
    oh = (batch_row[...] == lax.broadcasted_iota(jnp.int32, (G, N), 0)
          ).astype(jnp.float32)
    S = lax.dot_general(oh, h2, (((1,), (0,)), ((), ())),
                        preferred_element_type=jnp.float32, precision=_PREC)
    cnt = jnp.sum(oh, axis=1, keepdims=True)
    g = S / jnp.maximum(cnt, 1.0)
    mu = jnp.dot(g, wmu[...], preferred_element_type=jnp.float32,
                 precision=_PREC) + bmu[...]
    lv = jnp.dot(g, wlv[...], preferred_element_type=jnp.float32,
                 precision=_PREC) + blv[...]
    std = jnp.exp(0.5 * lv)
    z = mu + eps[...] * std
    hr = jnp.maximum(jnp.dot(z, wd1[...], preferred_element_type=jnp.float32,
                             precision=_PREC) + bd1[...], 0.0)
    recon_o[...] = jnp.dot(hr, wd2[...], preferred_element_type=jnp.float32,
                           precision=_PREC) + bd2[...]
    mu_o[...] = mu
    lv_o[...] = lv


_tc3 = pl.pallas_call(
    _tc3_body,
    out_shape=[jax.ShapeDtypeStruct((G, D), jnp.float32),
               jax.ShapeDtypeStruct((G, Z), jnp.float32),
               jax.ShapeDtypeStruct((G, Z), jnp.float32)],
)


def kernel(x, edge_index, batch, eps, W1, b1, W2, b2,
           Wmu, bmu, Wlv, blv, Wd1, bd1, Wd2, bd2):
    src = edge_index[0].astype(jnp.int32)
    dst = edge_index[1].astype(jnp.int32)
    padn = jnp.full((EP - E,), N, jnp.int32)
    srcp = jnp.concatenate([src, padn]).reshape(EP // CW, CW)
    dstp = jnp.concatenate([dst, padn]).reshape(EP // CW, CW)

    deg_parts = _deg_kernel(dstp)
    degT = deg_parts.T[:N]
    u1, dinv = _tc1(degT, x, W1)

    zpad = jnp.zeros((NU - N, H), jnp.float32)
    agg1 = _agg_kernel(jnp.concatenate([u1, zpad]), srcp, dstp)
    u2 = _tc2(agg1, u1, dinv, b1.reshape(1, H), W2)
    agg2 = _agg_kernel(jnp.concatenate([u2, zpad]), srcp, dstp)

    recon, mu, lv = _tc3(
        agg2, u2, dinv, b2.reshape(1, H),
        batch.astype(jnp.int32).reshape(1, N), eps,
        Wmu, bmu.reshape(1, Z), Wlv, blv.reshape(1, Z),
        Wd1, bd1.reshape(1, H), Wd2, bd2.reshape(1, D))
    return (recon, mu, lv)


# speedup vs baseline: 1.0885x; 1.0110x over previous
import jax
import jax.numpy as jnp
from jax import lax
from jax.experimental import pallas as pl
from jax.experimental.pallas import tpu as pltpu
from jax.experimental.pallas import tpu_sc as plsc

N = 10000
E = 320000
D = 128
H = 128
Z = 64
G = 64

NC, NS, L = 2, 16, 16
NTILES = NC * NS
CW = 128
RPT = 80
EP = NTILES * RPT * CW
NP = 10240
NACC = 10112
SEG = NACC // NS
CHI = 16
A0 = 8
A1 = 2
NU = N + 16

_mesh = plsc.VectorSubcoreMesh(
    core_axis_name="c", subcore_axis_name="s", num_cores=NC, num_subcores=NS)


def _deg_body(dst_hbm, out_hbm, dst_v, deg_v):
    c = lax.axis_index("c")
    s = lax.axis_index("s")
    wid = c * NS + s
    pltpu.sync_copy(dst_hbm.at[pl.ds(wid * RPT, RPT)], dst_v)
    zeros16 = jnp.zeros((L,), jnp.float32)

    def zero_body(i, carry):
        deg_v[pl.ds(i * L, L)] = zeros16
        return carry

    lax.fori_loop(0, NP // L, zero_body, 0)
    ones16 = jnp.ones((L,), jnp.float32)

    def edge_body(j, carry):
        for k in range(CW // L):
            idx = dst_v[j, pl.ds(k * L, L)]
            plsc.addupdate_scatter(deg_v, [idx], ones16)
        return carry

    lax.fori_loop(0, RPT, edge_body, 0)
    pltpu.sync_copy(deg_v, out_hbm.at[wid])


_deg_kernel = pl.kernel(
    _deg_body,
    out_type=jax.ShapeDtypeStruct((NTILES, NP), jnp.float32),
    mesh=_mesh,
    scratch_types=[
        pltpu.VMEM((RPT, CW), jnp.int32),
        pltpu.VMEM((NP,), jnp.float32),
    ],
    compiler_params=pltpu.CompilerParams(needs_layout_passes=False),
)


def _agg_body(u_hbm, src_hbm, dst_hbm, out_hbm,
              src_v, dst_v, bufa, bufb, acc_sh, sema, semb):
    c = lax.axis_index("c")
    s = lax.axis_index("s")
    zeros16 = jnp.zeros((L,), jnp.float32)

    def zb(r, carry):
        for k in range(H // L):
            bufa[r, pl.ds(k * L, L)] = zeros16
        return carry

    lax.fori_loop(0, CW, zb, 0)
    for k in range(SEG // CW):
        pltpu.sync_copy(bufa, acc_sh.at[pl.ds(s * SEG + k * CW, CW)])
    rem = SEG - (SEG // CW) * CW
    if rem:
        pltpu.sync_copy(bufa.at[pl.ds(0, rem)],
                        acc_sh.at[pl.ds(s * SEG + (SEG // CW) * CW, rem)])
    plsc.subcore_barrier()

    def step(j, buf, sem):
        pltpu.make_async_copy(u_hbm.at[src_v.at[j]], buf, sem).wait()
        pltpu.sync_copy(buf, acc_sh.at[dst_v.at[j]], add=True)

    nchunks = jnp.where(c == 0, A0, A1)
    base_chunk = jnp.where(c == 0, s * A0, NS * A0 + s * A1)

    def chunk_body(ci, carry):
        row0 = (base_chunk + ci) * CHI
        pltpu.sync_copy(src_hbm.at[pl.ds(row0, CHI)], src_v)
        pltpu.sync_copy(dst_hbm.at[pl.ds(row0, CHI)], dst_v)
        pltpu.async_copy(u_hbm.at[src_v.at[0]], bufa, sema)

        def body(jj, carry2):
            j0 = 2 * jj
            pltpu.async_copy(u_hbm.at[src_v.at[j0 + 1]], bufb, semb)
            step(j0, bufa, sema)
            pltpu.async_copy(u_hbm.at[src_v.at[j0 + 2]], bufa, sema)
            step(j0 + 1, bufb, semb)
            return carry2

        lax.fori_loop(0, CHI // 2 - 1, body, 0)
        pltpu.async_copy(u_hbm.at[src_v.at[CHI - 1]], bufb, semb)
        step(CHI - 2, bufa, sema)
        step(CHI - 1, bufb, semb)
        return carry

    lax.fori_loop(0, nchunks, chunk_body, 0)
    plsc.subcore_barrier()
    pltpu.sync_copy(acc_sh.at[pl.ds(s * SEG, SEG)],
                    out_hbm.at[c, pl.ds(s * SEG, SEG)])


_agg_kernel = pl.kernel(
    _agg_body,
    out_type=jax.ShapeDtypeStruct((NC, NACC, H), jnp.float32),
    mesh=_mesh,
    scratch_types=[
        pltpu.VMEM((CHI, CW), jnp.int32),
        pltpu.VMEM((CHI, CW), jnp.int32),
        pltpu.VMEM((CW, H), jnp.float32),
        pltpu.VMEM((CW, H), jnp.float32),
        pltpu.VMEM_SHARED((NACC, H), jnp.float32),
        pltpu.SemaphoreType.DMA,
        pltpu.SemaphoreType.DMA,
    ],
    compiler_params=pltpu.CompilerParams(needs_layout_passes=False),
)

_PREC = lax.Precision.HIGHEST


def _tc1_body(degT, x, w1, u1_o, dinv_o):
    deg = jnp.sum(degT[...], axis=1, keepdims=True) + 1.0
    dinv = 1.0 / jnp.sqrt(deg)
    dinv_o[...] = dinv
    u1_o[...] = jnp.dot(x[...] * dinv, w1[...],
                        preferred_element_type=jnp.float32, precision=_PREC)


_tc1 = pl.pallas_call(
    _tc1_body,
    out_shape=[jax.ShapeDtypeStruct((N, H), jnp.float32),
               jax.ShapeDtypeStruct((N, 1), jnp.float32)],
)


def _tc2_body(parts, u1, dinv, b1, w2, u2_o):
    agg = parts[0, :N, :] + parts[1, :N, :] + u1[...]
    h1 = jnp.maximum(dinv[...] * agg + b1[...], 0.0)
    u2_o[...] = jnp.dot(h1 * dinv[...], w2[...],
                        preferred_element_type=jnp.float32, precision=_PREC)


_tc2 = pl.pallas_call(
    _tc2_body,
    out_shape=jax.ShapeDtypeStruct((N, H), jnp.float32),
)


def _tc3_body(parts, u2, dinv, b2, batch_row, eps,
              wmu, bmu, wlv, blv, wd1, bd1, wd2, bd2,
              recon_o, mu_o, lv_o):
    agg = parts[0, :N, :] + parts[1, :N, :] + u2[...]
    h2 = jnp.maximum(dinv[...] * agg + b2[...], 0.0)
    oh = (batch_row[...] == lax.broadcasted_iota(jnp.int32, (G, N), 0)
          ).astype(jnp.float32)
    S = lax.dot_general(oh, h2, (((1,), (0,)), ((), ())),
                        preferred_element_type=jnp.float32, precision=_PREC)
    cnt = jnp.sum(oh, axis=1, keepdims=True)
    g = S / jnp.maximum(cnt, 1.0)
    mu = jnp.dot(g, wmu[...], preferred_element_type=jnp.float32,
                 precision=_PREC) + bmu[...]
    lv = jnp.dot(g, wlv[...], preferred_element_type=jnp.float32,
                 precision=_PREC) + blv[...]
    std = jnp.exp(0.5 * lv)
    z = mu + eps[...] * std
    hr = jnp.maximum(jnp.dot(z, wd1[...], preferred_element_type=jnp.float32,
                             precision=_PREC) + bd1[...], 0.0)
    recon_o[...] = jnp.dot(hr, wd2[...], preferred_element_type=jnp.float32,
                           precision=_PREC) + bd2[...]
    mu_o[...] = mu
    lv_o[...] = lv


_tc3 = pl.pallas_call(
    _tc3_body,
    out_shape=[jax.ShapeDtypeStruct((G, D), jnp.float32),
               jax.ShapeDtypeStruct((G, Z), jnp.float32),
               jax.ShapeDtypeStruct((G, Z), jnp.float32)],
)


def kernel(x, edge_index, batch, eps, W1, b1, W2, b2,
           Wmu, bmu, Wlv, blv, Wd1, bd1, Wd2, bd2):
    src = edge_index[0].astype(jnp.int32)
    dst = edge_index[1].astype(jnp.int32)
    padn = jnp.full((EP - E,), N, jnp.int32)
    srcp = jnp.concatenate([src, padn]).reshape(EP // CW, CW)
    dstp = jnp.concatenate([dst, padn]).reshape(EP // CW, CW)

    deg_parts = _deg_kernel(dstp)
    degT = deg_parts.T[:N]
    u1, dinv = _tc1(degT, x, W1)

    zpad = jnp.zeros((NU - N, H), jnp.float32)
    agg1 = _agg_kernel(jnp.concatenate([u1, zpad]), srcp, dstp)
    u2 = _tc2(agg1, u1, dinv, b1.reshape(1, H), W2)
    agg2 = _agg_kernel(jnp.concatenate([u2, zpad]), srcp, dstp)

    recon, mu, lv = _tc3(
        agg2, u2, dinv, b2.reshape(1, H),
        batch.astype(jnp.int32).reshape(1, N), eps,
        Wmu, bmu.reshape(1, Z), Wlv, blv.reshape(1, Z),
        Wd1, bd1.reshape(1, H), Wd2, bd2.reshape(1, D))
    return (recon, mu, lv)
